# Initial kernel scaffold; baseline (speedup 1.0000x reference)
import jax
import jax.numpy as jnp
from jax.experimental import pallas as pl


def kernel(atom_features, bond_features, pair_indices, W_lin, b_lin, W_ih, W_hh, b_ih, b_hh):
    raise NotImplementedError("write your pallas kernel here")


# speedup vs baseline: 1.2293x; 1.2293x over previous
import functools

import jax
import jax.numpy as jnp
from jax import lax
from jax.experimental import pallas as pl
from jax.experimental.pallas import tpu as pltpu
from jax.experimental.pallas import tpu_sc as plsc

D = 32
BD = 16
KA = BD + 1
N_ATOMS = 100000
N_EDGES = 200000
N_STEPS = 4

NC, NS = 2, 16
NW = NC * NS

E_PAD = 204800
EW = E_PAD // NW
CG = 1280
RG = CG // 128
OC_G = EW // CG

ET = E_PAD // NS
CS = 1280
RS = CS // 128
OC_S = ET // CS

NPHASE = 2
RSZ = 25088
SPM_ROWS = RSZ + 128
ZT = SPM_ROWS // NS
WT = RSZ // NS

EB = 1024
NB = 2000


@functools.lru_cache(maxsize=None)
def _sc_kernels():
    mesh = plsc.VectorSubcoreMesh(
        core_axis_name="c", subcore_axis_name="s",
        num_cores=NC, num_subcores=NS)
    sc_params = pltpu.CompilerParams(use_tc_tiling_on_sc=False)

    @functools.partial(
        pl.kernel,
        out_type=jax.ShapeDtypeStruct((E_PAD, D), jnp.float32),
        mesh=mesh,
        compiler_params=sc_params,
        scratch_types=[
            pltpu.VMEM((CG,), jnp.int32),
            pltpu.VMEM((CG, D), jnp.float32),
            pltpu.SemaphoreType.DMA,
        ],
    )
    def sc_gather(h_hbm, dst_hbm, out_hbm, idx_v, rows_v, sem):
        wid = lax.axis_index("s") * NC + lax.axis_index("c")

        def round_body(oc, carry):
            base = wid * EW + oc * CG
            pltpu.sync_copy(dst_hbm.at[pl.ds(base, CG)], idx_v)
            cps = [
                pltpu.async_copy(h_hbm.at[idx_v.at[pl.ds(j * 128, 128)]],
                                 rows_v.at[pl.ds(j * 128, 128)], sem)
                for j in range(RG)
            ]
            for cp in cps:
                cp.wait()
            pltpu.sync_copy(rows_v, out_hbm.at[pl.ds(base, CG)])
            return carry

        lax.fori_loop(0, OC_G, round_body, 0)

    @functools.partial(
        pl.kernel,
        out_type=jax.ShapeDtypeStruct((NPHASE * NC, RSZ, D), jnp.float32),
        mesh=mesh,
        compiler_params=sc_params,
        scratch_types=[
            pltpu.VMEM((CS,), jnp.int32),
            pltpu.VMEM((RS, 128), jnp.int32),
            pltpu.VMEM((CS, D), jnp.float32),
            pltpu.VMEM_SHARED((SPM_ROWS, D), jnp.float32),
        ],
    )
    def sc_scatter(t_hbm, src_hbm, zeros_hbm, out_hbm, sidx_v, lidx_v,
                   rows_v, shared):
        core = lax.axis_index("c")
        tid = lax.axis_index("s")

        for phase in range(NPHASE):
            rng = phase * NC + core
            lo = rng * RSZ

            pltpu.sync_copy(zeros_hbm.at[pl.ds(tid * ZT, ZT)],
                            shared.at[pl.ds(tid * ZT, ZT)])
            plsc.subcore_barrier()

            def round_body(oc, carry):
                ebase = tid * ET + oc * CS
                pltpu.sync_copy(src_hbm.at[pl.ds(ebase, CS)], sidx_v)
                pltpu.sync_copy(t_hbm.at[pl.ds(ebase, CS)], rows_v)

                def idx_body(i, carry2):
                    v = sidx_v[pl.ds(i * 16, 16)]
                    li = v - lo
                    ok = (li >= 0) & (li < RSZ)
                    lidx_v[i // 8, pl.ds((i % 8) * 16, 16)] = jnp.where(
                        ok, li, RSZ)
                    return carry2

                lax.fori_loop(0, RS * 8, idx_body, 0)
                for j in range(RS):
                    pltpu.sync_copy(rows_v.at[pl.ds(j * 128, 128)],
                                    shared.at[lidx_v.at[j]], add=True)
                return carry

            lax.fori_loop(0, OC_S, round_body, 0)
            plsc.subcore_barrier()
            pltpu.sync_copy(shared.at[pl.ds(tid * WT, WT)],
                            out_hbm.at[rng, pl.ds(tid * WT, WT)])
            plsc.subcore_barrier()

    return sc_gather, sc_scatter


def _bf(x):
    return x.astype(jnp.bfloat16).astype(jnp.float32)


def _transform_body(nbr_ref, ba_ref, w_ref, bl_ref, s_ref, out_ref):
    bm = lax.dot_general(ba_ref[...], w_ref[...], (((1,), (0,)), ((), ())),
                         preferred_element_type=jnp.float32) + bl_ref[...]
    z = _bf(bm) * jnp.tile(_bf(nbr_ref[...]), (1, D))
    out_ref[...] = lax.dot_general(z, s_ref[...], (((1,), (0,)), ((), ())),
                                   precision=lax.Precision.HIGHEST,
                                   preferred_element_type=jnp.float32)


def _tc_transform(nbr, bond16, w2, blin_row, sel, interpret=False):
    return pl.pallas_call(
        _transform_body,
        grid=(E_PAD // EB,),
        in_specs=[
            pl.BlockSpec((EB, D), lambda i: (i, 0)),
            pl.BlockSpec((EB, BD), lambda i: (i, 0)),
            pl.BlockSpec((BD, D * D), lambda i: (0, 0)),
            pl.BlockSpec((1, D * D), lambda i: (0, 0)),
            pl.BlockSpec((D * D, D), lambda i: (0, 0)),
        ],
        out_specs=pl.BlockSpec((EB, D), lambda i: (i, 0)),
        out_shape=jax.ShapeDtypeStruct((E_PAD, D), jnp.float32),
        interpret=interpret,
    )(nbr, bond16, w2, blin_row, sel)


def _exp_precise(x):
    x = jnp.clip(x, -87.0, 87.0)
    k = jnp.round(x * 1.4426950408889634)
    t = x - k * 0.6931471805599453
    p = 1.0 + t * (1.0 + t * (0.5 + t * (
        0.16666666666666666 + t * (0.041666666666666664 + t * (
            0.008333333333333333 + t * (
                0.001388888888888889 + t * 0.0001984126984126984))))))
    scale = lax.bitcast_convert_type(
        (k.astype(jnp.int32) + 127) << 23, jnp.float32)
    return p * scale


def _sigmoid(x):
    return 1.0 / (1.0 + _exp_precise(-x))


def _tanh(x):
    t = _exp_precise(-2.0 * jnp.abs(x))
    return jnp.sign(x) * (1.0 - t) / (1.0 + t)


def _gru_body(a_ref, h_ref, wi_ref, wh_ref, bi_ref, bh_ref, out_ref):
    a = a_ref[...]
    h = h_ref[...]
    gi = lax.dot_general(_bf(a), wi_ref[...], (((1,), (0,)), ((), ())),
                         preferred_element_type=jnp.float32) + bi_ref[...]
    gh = lax.dot_general(_bf(h), wh_ref[...], (((1,), (0,)), ((), ())),
                         preferred_element_type=jnp.float32) + bh_ref[...]
    i_r, i_z, i_n = gi[:, 0:D], gi[:, D:2 * D], gi[:, 2 * D:3 * D]
    h_r, h_z, h_n = gh[:, 0:D], gh[:, D:2 * D], gh[:, 2 * D:3 * D]
    r = _sigmoid(i_r + h_r)
    z = _sigmoid(i_z + h_z)
    n = _tanh(i_n + r * h_n)
    out_ref[...] = (1.0 - z) * n + z * h


def _tc_gru(agg, h, wi, wh, bi, bh, interpret=False):
    return pl.pallas_call(
        _gru_body,
        grid=(N_ATOMS // NB,),
        in_specs=[
            pl.BlockSpec((NB, D), lambda i: (i, 0)),
            pl.BlockSpec((NB, D), lambda i: (i, 0)),
            pl.BlockSpec((D, 3 * D), lambda i: (0, 0)),
            pl.BlockSpec((D, 3 * D), lambda i: (0, 0)),
            pl.BlockSpec((1, 3 * D), lambda i: (0, 0)),
            pl.BlockSpec((1, 3 * D), lambda i: (0, 0)),
        ],
        out_specs=pl.BlockSpec((NB, D), lambda i: (i, 0)),
        out_shape=jax.ShapeDtypeStruct((N_ATOMS, D), jnp.float32),
        interpret=interpret,
    )(agg, h, wi, wh, bi, bh)


def kernel(atom_features, bond_features, pair_indices, W_lin, b_lin,
           W_ih, W_hh, b_ih, b_hh):
    sc_gather, sc_scatter = _sc_kernels()
    src = pair_indices[:, 0]
    dst = pair_indices[:, 1]
    npad = E_PAD - N_EDGES
    dst_pad = jnp.concatenate([dst, jnp.zeros((npad,), jnp.int32)])
    src_pad = jnp.concatenate([src, jnp.full((npad,), 1 << 28, jnp.int32)])
    bond16 = jnp.concatenate([
        _bf(bond_features), jnp.zeros((npad, BD), jnp.float32)], axis=0)
    w2 = _bf(W_lin).T
    blin_row = b_lin.reshape(1, D * D)
    sel = jnp.repeat(jnp.eye(D, dtype=jnp.float32), D, axis=0)
    wi = _bf(W_ih.T)
    wh = _bf(W_hh.T)
    bi = b_ih.reshape(1, 3 * D)
    bh = b_hh.reshape(1, 3 * D)
    zeros_buf = jnp.zeros((SPM_ROWS, D), jnp.float32)

    h = atom_features
    for _ in range(N_STEPS):
        nbr = sc_gather(h, dst_pad)
        transformed = _tc_transform(nbr, bond16, w2, blin_row, sel)
        agg4 = sc_scatter(transformed, src_pad, zeros_buf)
        agg = agg4.reshape(NPHASE * NC * RSZ, D)[:N_ATOMS]
        h = _tc_gru(agg, h, wi, wh, bi, bh)
    return h


# speedup vs baseline: 1.4415x; 1.1726x over previous
import functools

import jax
import jax.numpy as jnp
from jax import lax
from jax.experimental import pallas as pl
from jax.experimental.pallas import tpu as pltpu
from jax.experimental.pallas import tpu_sc as plsc

D = 32
BD = 16
KA = BD + 1
N_ATOMS = 100000
N_EDGES = 200000
N_STEPS = 4

NC, NS = 2, 16
NW = NC * NS

E_PAD = 204800
EW = E_PAD // NW
CG = 1280
RG = CG // 128
OC_G = EW // CG

ET = E_PAD // NS
CS = 1280
RS = CS // 128
OC_S = ET // CS

NPHASE = 2
RSZ = 25088
SPM_ROWS = RSZ + 128
ZT = SPM_ROWS // NS
WT = RSZ // NS

EB = 1024
NB = 2000


@functools.lru_cache(maxsize=None)
def _sc_kernels():
    mesh = plsc.VectorSubcoreMesh(
        core_axis_name="c", subcore_axis_name="s",
        num_cores=NC, num_subcores=NS)
    sc_params = pltpu.CompilerParams(use_tc_tiling_on_sc=False)

    @functools.partial(
        pl.kernel,
        out_type=jax.ShapeDtypeStruct((E_PAD, D), jnp.float32),
        mesh=mesh,
        compiler_params=sc_params,
        scratch_types=[
            pltpu.VMEM((CG,), jnp.int32),
            pltpu.VMEM((CG, D), jnp.float32),
            pltpu.SemaphoreType.DMA,
        ],
    )
    def sc_gather(h_hbm, dst_hbm, out_hbm, idx_v, rows_v, sem):
        wid = lax.axis_index("s") * NC + lax.axis_index("c")

        def round_body(oc, carry):
            base = wid * EW + oc * CG
            pltpu.sync_copy(dst_hbm.at[pl.ds(base, CG)], idx_v)
            cps = [
                pltpu.async_copy(h_hbm.at[idx_v.at[pl.ds(j * 128, 128)]],
                                 rows_v.at[pl.ds(j * 128, 128)], sem)
                for j in range(RG)
            ]
            for cp in cps:
                cp.wait()
            pltpu.sync_copy(rows_v, out_hbm.at[pl.ds(base, CG)])
            return carry

        lax.fori_loop(0, OC_G, round_body, 0)

    @functools.partial(
        pl.kernel,
        out_type=jax.ShapeDtypeStruct((NPHASE * NC, RSZ, D), jnp.float32),
        mesh=mesh,
        compiler_params=sc_params,
        scratch_types=[
            pltpu.VMEM((CS,), jnp.int32),
            pltpu.VMEM((RS, 128), jnp.int32),
            pltpu.VMEM((CS, D), jnp.float32),
            pltpu.VMEM_SHARED((SPM_ROWS, D), jnp.float32),
        ],
    )
    def sc_scatter(t_hbm, src_hbm, zeros_hbm, out_hbm, sidx_v, lidx_v,
                   rows_v, shared):
        core = lax.axis_index("c")
        tid = lax.axis_index("s")

        for phase in range(NPHASE):
            rng = phase * NC + core
            lo = rng * RSZ

            pltpu.sync_copy(zeros_hbm.at[pl.ds(tid * ZT, ZT)],
                            shared.at[pl.ds(tid * ZT, ZT)])
            plsc.subcore_barrier()

            def round_body(oc, carry):
                ebase = tid * ET + oc * CS
                pltpu.sync_copy(src_hbm.at[pl.ds(ebase, CS)], sidx_v)
                pltpu.sync_copy(t_hbm.at[pl.ds(ebase, CS)], rows_v)

                def idx_body(i, carry2):
                    v = sidx_v[pl.ds(i * 16, 16)]
                    li = v - lo
                    ok = (li >= 0) & (li < RSZ)
                    dummy = RSZ + (v & 127)
                    lidx_v[i // 8, pl.ds((i % 8) * 16, 16)] = jnp.where(
                        ok, li, dummy)
                    return carry2

                lax.fori_loop(0, RS * 8, idx_body, 0)
                for j in range(RS):
                    pltpu.sync_copy(rows_v.at[pl.ds(j * 128, 128)],
                                    shared.at[lidx_v.at[j]], add=True)
                return carry

            lax.fori_loop(0, OC_S, round_body, 0)
            plsc.subcore_barrier()
            pltpu.sync_copy(shared.at[pl.ds(tid * WT, WT)],
                            out_hbm.at[rng, pl.ds(tid * WT, WT)])
            plsc.subcore_barrier()

    return sc_gather, sc_scatter


def _bf(x):
    return x.astype(jnp.bfloat16).astype(jnp.float32)


def _transform_body(nbr_ref, ba_ref, w_ref, bl_ref, s_ref, out_ref):
    bm = lax.dot_general(ba_ref[...], w_ref[...], (((1,), (0,)), ((), ())),
                         preferred_element_type=jnp.float32) + bl_ref[...]
    z = _bf(bm) * jnp.tile(_bf(nbr_ref[...]), (1, D))
    out_ref[...] = lax.dot_general(z, s_ref[...], (((1,), (0,)), ((), ())),
                                   precision=lax.Precision.HIGHEST,
                                   preferred_element_type=jnp.float32)


def _tc_transform(nbr, bond16, w2, blin_row, sel, interpret=False):
    return pl.pallas_call(
        _transform_body,
        grid=(E_PAD // EB,),
        in_specs=[
            pl.BlockSpec((EB, D), lambda i: (i, 0)),
            pl.BlockSpec((EB, BD), lambda i: (i, 0)),
            pl.BlockSpec((BD, D * D), lambda i: (0, 0)),
            pl.BlockSpec((1, D * D), lambda i: (0, 0)),
            pl.BlockSpec((D * D, D), lambda i: (0, 0)),
        ],
        out_specs=pl.BlockSpec((EB, D), lambda i: (i, 0)),
        out_shape=jax.ShapeDtypeStruct((E_PAD, D), jnp.float32),
        interpret=interpret,
    )(nbr, bond16, w2, blin_row, sel)


def _exp_precise(x):
    x = jnp.clip(x, -87.0, 87.0)
    k = jnp.round(x * 1.4426950408889634)
    t = x - k * 0.6931471805599453
    p = 1.0 + t * (1.0 + t * (0.5 + t * (
        0.16666666666666666 + t * (0.041666666666666664 + t * (
            0.008333333333333333 + t * (
                0.001388888888888889 + t * 0.0001984126984126984))))))
    scale = lax.bitcast_convert_type(
        (k.astype(jnp.int32) + 127) << 23, jnp.float32)
    return p * scale


def _sigmoid(x):
    return 1.0 / (1.0 + _exp_precise(-x))


def _tanh(x):
    t = _exp_precise(-2.0 * jnp.abs(x))
    return jnp.sign(x) * (1.0 - t) / (1.0 + t)


def _gru_body(a_ref, h_ref, wi_ref, wh_ref, bi_ref, bh_ref, out_ref):
    a = a_ref[...]
    h = h_ref[...]
    gi = lax.dot_general(_bf(a), wi_ref[...], (((1,), (0,)), ((), ())),
                         preferred_element_type=jnp.float32) + bi_ref[...]
    gh = lax.dot_general(_bf(h), wh_ref[...], (((1,), (0,)), ((), ())),
                         preferred_element_type=jnp.float32) + bh_ref[...]
    i_r, i_z, i_n = gi[:, 0:D], gi[:, D:2 * D], gi[:, 2 * D:3 * D]
    h_r, h_z, h_n = gh[:, 0:D], gh[:, D:2 * D], gh[:, 2 * D:3 * D]
    r = _sigmoid(i_r + h_r)
    z = _sigmoid(i_z + h_z)
    n = _tanh(i_n + r * h_n)
    out_ref[...] = (1.0 - z) * n + z * h


def _tc_gru(agg, h, wi, wh, bi, bh, interpret=False):
    return pl.pallas_call(
        _gru_body,
        grid=(N_ATOMS // NB,),
        in_specs=[
            pl.BlockSpec((NB, D), lambda i: (i, 0)),
            pl.BlockSpec((NB, D), lambda i: (i, 0)),
            pl.BlockSpec((D, 3 * D), lambda i: (0, 0)),
            pl.BlockSpec((D, 3 * D), lambda i: (0, 0)),
            pl.BlockSpec((1, 3 * D), lambda i: (0, 0)),
            pl.BlockSpec((1, 3 * D), lambda i: (0, 0)),
        ],
        out_specs=pl.BlockSpec((NB, D), lambda i: (i, 0)),
        out_shape=jax.ShapeDtypeStruct((N_ATOMS, D), jnp.float32),
        interpret=interpret,
    )(agg, h, wi, wh, bi, bh)


def kernel(atom_features, bond_features, pair_indices, W_lin, b_lin,
           W_ih, W_hh, b_ih, b_hh):
    sc_gather, sc_scatter = _sc_kernels()
    src = pair_indices[:, 0]
    dst = pair_indices[:, 1]
    npad = E_PAD - N_EDGES
    dst_pad = jnp.concatenate([dst, jnp.zeros((npad,), jnp.int32)])
    src_pad = jnp.concatenate([src, jnp.full((npad,), 1 << 28, jnp.int32)])
    bond16 = jnp.concatenate([
        _bf(bond_features), jnp.zeros((npad, BD), jnp.float32)], axis=0)
    w2 = _bf(W_lin).T
    blin_row = b_lin.reshape(1, D * D)
    sel = jnp.repeat(jnp.eye(D, dtype=jnp.float32), D, axis=0)
    wi = _bf(W_ih.T)
    wh = _bf(W_hh.T)
    bi = b_ih.reshape(1, 3 * D)
    bh = b_hh.reshape(1, 3 * D)
    zeros_buf = jnp.zeros((SPM_ROWS, D), jnp.float32)

    h = atom_features
    for _ in range(N_STEPS):
        nbr = sc_gather(h, dst_pad)
        transformed = _tc_transform(nbr, bond16, w2, blin_row, sel)
        agg4 = sc_scatter(transformed, src_pad, zeros_buf)
        agg = agg4.reshape(NPHASE * NC * RSZ, D)[:N_ATOMS]
        h = _tc_gru(agg, h, wi, wh, bi, bh)
    return h


# speedup vs baseline: 1.9632x; 1.3619x over previous
import functools

import jax
import jax.numpy as jnp
from jax import lax
from jax.experimental import pallas as pl
from jax.experimental.pallas import tpu as pltpu
from jax.experimental.pallas import tpu_sc as plsc

D = 32
BD = 16
KA = BD + 1
N_ATOMS = 100000
N_EDGES = 200000
N_STEPS = 4

NC, NS = 2, 16
NW = NC * NS

E_PAD = 204800
EW = E_PAD // NW
CG = 1280
RG = CG // 128
OC_G = EW // CG

ET = E_PAD // NS
CS = 1280
RS = CS // 128
OC_S = ET // CS

NPHASE = 2
RSZ = 25088
SPM_ROWS = RSZ + 128
ZT = SPM_ROWS // NS
WT = RSZ // NS

EB = 1024
NB = 2000


@functools.lru_cache(maxsize=None)
def _sc_kernels():
    mesh = plsc.VectorSubcoreMesh(
        core_axis_name="c", subcore_axis_name="s",
        num_cores=NC, num_subcores=NS)
    sc_params = pltpu.CompilerParams(use_tc_tiling_on_sc=False)

    @functools.partial(
        pl.kernel,
        out_type=jax.ShapeDtypeStruct((E_PAD, D), jnp.float32),
        mesh=mesh,
        compiler_params=sc_params,
        scratch_types=[
            pltpu.VMEM((CG,), jnp.int32),
            pltpu.VMEM((CG, D), jnp.float32),
            pltpu.SemaphoreType.DMA,
        ],
    )
    def sc_gather(h_hbm, dst_hbm, out_hbm, idx_v, rows_v, sem):
        wid = lax.axis_index("s") * NC + lax.axis_index("c")

        def round_body(oc, carry):
            base = wid * EW + oc * CG
            pltpu.sync_copy(dst_hbm.at[pl.ds(base, CG)], idx_v)
            cps = [
                pltpu.async_copy(h_hbm.at[idx_v.at[pl.ds(j * 128, 128)]],
                                 rows_v.at[pl.ds(j * 128, 128)], sem)
                for j in range(RG)
            ]
            for cp in cps:
                cp.wait()
            pltpu.sync_copy(rows_v, out_hbm.at[pl.ds(base, CG)])
            return carry

        lax.fori_loop(0, OC_G, round_body, 0)

    @functools.partial(
        pl.kernel,
        out_type=jax.ShapeDtypeStruct((NPHASE * NC, RSZ, D), jnp.float32),
        mesh=mesh,
        compiler_params=sc_params,
        scratch_types=[
            pltpu.VMEM((CS,), jnp.int32),
            pltpu.VMEM((RS, 128), jnp.int32),
            pltpu.VMEM((CS, D), jnp.float32),
            pltpu.VMEM_SHARED((SPM_ROWS, D), jnp.float32),
        ],
    )
    def sc_scatter(t_hbm, src_hbm, zeros_hbm, out_hbm, sidx_v, lidx_v,
                   rows_v, shared):
        core = lax.axis_index("c")
        tid = lax.axis_index("s")

        for phase in range(NPHASE):
            rng = phase * NC + core
            lo = rng * RSZ

            pltpu.sync_copy(zeros_hbm.at[pl.ds(tid * ZT, ZT)],
                            shared.at[pl.ds(tid * ZT, ZT)])
            plsc.subcore_barrier()

            def round_body(oc, carry):
                ebase = tid * ET + oc * CS
                pltpu.sync_copy(src_hbm.at[pl.ds(ebase, CS)], sidx_v)
                pltpu.sync_copy(t_hbm.at[pl.ds(ebase, CS)], rows_v)

                def idx_body(i, carry2):
                    v = sidx_v[pl.ds(i * 16, 16)]
                    li = v - lo
                    ok = (li >= 0) & (li < RSZ)
                    dummy = RSZ + (v & 127)
                    lidx_v[i // 8, pl.ds((i % 8) * 16, 16)] = jnp.where(
                        ok, li, dummy)
                    return carry2

                lax.fori_loop(0, RS * 8, idx_body, 0)
                for j in range(RS):
                    pltpu.sync_copy(rows_v.at[pl.ds(j * 128, 128)],
                                    shared.at[lidx_v.at[j]], add=True)
                return carry

            lax.fori_loop(0, OC_S, round_body, 0)
            plsc.subcore_barrier()
            pltpu.sync_copy(shared.at[pl.ds(tid * WT, WT)],
                            out_hbm.at[rng, pl.ds(tid * WT, WT)])
            plsc.subcore_barrier()

    return sc_gather, sc_scatter


def _bf(x):
    return x.astype(jnp.bfloat16).astype(jnp.float32)


def _transform_body(nbr_ref, ba_ref, w_ref, bl_ref, s_ref, out_ref):
    dn = (((1,), (0,)), ((), ()))
    bm = lax.dot_general(ba_ref[...], w_ref[...], dn,
                         preferred_element_type=jnp.float32) + bl_ref[...]
    z = _bf(bm) * jnp.tile(_bf(nbr_ref[...]), (1, D))
    zh = z.astype(jnp.bfloat16)
    zl = (z - zh.astype(jnp.float32)).astype(jnp.bfloat16)
    s = s_ref[...]
    out_ref[...] = (
        lax.dot_general(zh, s, dn, preferred_element_type=jnp.float32)
        + lax.dot_general(zl, s, dn, preferred_element_type=jnp.float32))


def _tc_transform(nbr, bond16, w2, blin_row, sel, interpret=False):
    return pl.pallas_call(
        _transform_body,
        grid=(E_PAD // EB,),
        in_specs=[
            pl.BlockSpec((EB, D), lambda i: (i, 0)),
            pl.BlockSpec((EB, BD), lambda i: (i, 0)),
            pl.BlockSpec((BD, D * D), lambda i: (0, 0)),
            pl.BlockSpec((1, D * D), lambda i: (0, 0)),
            pl.BlockSpec((D * D, D), lambda i: (0, 0)),
        ],
        out_specs=pl.BlockSpec((EB, D), lambda i: (i, 0)),
        out_shape=jax.ShapeDtypeStruct((E_PAD, D), jnp.float32),
        interpret=interpret,
    )(nbr, bond16, w2, blin_row, sel)


def _exp_precise(x):
    x = jnp.clip(x, -87.0, 87.0)
    k = jnp.round(x * 1.4426950408889634)
    t = x - k * 0.6931471805599453
    p = 1.0 + t * (1.0 + t * (0.5 + t * (
        0.16666666666666666 + t * (0.041666666666666664 + t * (
            0.008333333333333333 + t * (
                0.001388888888888889 + t * 0.0001984126984126984))))))
    scale = lax.bitcast_convert_type(
        (k.astype(jnp.int32) + 127) << 23, jnp.float32)
    return p * scale


def _sigmoid(x):
    return 1.0 / (1.0 + _exp_precise(-x))


def _tanh(x):
    t = _exp_precise(-2.0 * jnp.abs(x))
    return jnp.sign(x) * (1.0 - t) / (1.0 + t)


def _gru_body(a_ref, h_ref, wi_ref, wh_ref, bi_ref, bh_ref, out_ref):
    a = a_ref[...]
    h = h_ref[...]
    gi = lax.dot_general(a.astype(jnp.bfloat16), wi_ref[...],
                         (((1,), (0,)), ((), ())),
                         preferred_element_type=jnp.float32) + bi_ref[...]
    gh = lax.dot_general(h.astype(jnp.bfloat16), wh_ref[...],
                         (((1,), (0,)), ((), ())),
                         preferred_element_type=jnp.float32) + bh_ref[...]
    i_r, i_z, i_n = gi[:, 0:D], gi[:, D:2 * D], gi[:, 2 * D:3 * D]
    h_r, h_z, h_n = gh[:, 0:D], gh[:, D:2 * D], gh[:, 2 * D:3 * D]
    r = _sigmoid(i_r + h_r)
    z = _sigmoid(i_z + h_z)
    n = _tanh(i_n + r * h_n)
    out_ref[...] = (1.0 - z) * n + z * h


def _tc_gru(agg, h, wi, wh, bi, bh, interpret=False):
    return pl.pallas_call(
        _gru_body,
        grid=(N_ATOMS // NB,),
        in_specs=[
            pl.BlockSpec((NB, D), lambda i: (i, 0)),
            pl.BlockSpec((NB, D), lambda i: (i, 0)),
            pl.BlockSpec((D, 3 * D), lambda i: (0, 0)),
            pl.BlockSpec((D, 3 * D), lambda i: (0, 0)),
            pl.BlockSpec((1, 3 * D), lambda i: (0, 0)),
            pl.BlockSpec((1, 3 * D), lambda i: (0, 0)),
        ],
        out_specs=pl.BlockSpec((NB, D), lambda i: (i, 0)),
        out_shape=jax.ShapeDtypeStruct((N_ATOMS, D), jnp.float32),
        interpret=interpret,
    )(agg, h, wi, wh, bi, bh)


def kernel(atom_features, bond_features, pair_indices, W_lin, b_lin,
           W_ih, W_hh, b_ih, b_hh):
    sc_gather, sc_scatter = _sc_kernels()
    src = pair_indices[:, 0]
    dst = pair_indices[:, 1]
    npad = E_PAD - N_EDGES
    dst_pad = jnp.concatenate([dst, jnp.zeros((npad,), jnp.int32)])
    src_pad = jnp.concatenate([src, jnp.full((npad,), 1 << 28, jnp.int32)])
    bond16 = jnp.concatenate([
        bond_features.astype(jnp.bfloat16),
        jnp.zeros((npad, BD), jnp.bfloat16)], axis=0)
    w2 = W_lin.astype(jnp.bfloat16).T
    blin_row = b_lin.reshape(1, D * D)
    sel = jnp.repeat(jnp.eye(D, dtype=jnp.bfloat16), D, axis=0)
    wi = W_ih.T.astype(jnp.bfloat16)
    wh = W_hh.T.astype(jnp.bfloat16)
    bi = b_ih.reshape(1, 3 * D)
    bh = b_hh.reshape(1, 3 * D)
    zeros_buf = jnp.zeros((SPM_ROWS, D), jnp.float32)

    h = atom_features
    for _ in range(N_STEPS):
        nbr = sc_gather(h, dst_pad)
        transformed = _tc_transform(nbr, bond16, w2, blin_row, sel)
        agg4 = sc_scatter(transformed, src_pad, zeros_buf)
        agg = agg4.reshape(NPHASE * NC * RSZ, D)[:N_ATOMS]
        h = _tc_gru(agg, h, wi, wh, bi, bh)
    return h


# speedup vs baseline: 2.0158x; 1.0268x over previous
import functools

import jax
import jax.numpy as jnp
from jax import lax
from jax.experimental import pallas as pl
from jax.experimental.pallas import tpu as pltpu
from jax.experimental.pallas import tpu_sc as plsc

D = 32
BD = 16
KA = BD + 1
N_ATOMS = 100000
N_EDGES = 200000
N_STEPS = 4

NC, NS = 2, 16
NW = NC * NS

E_PAD = 204800
EW = E_PAD // NW
CG = 1280
RG = CG // 128
OC_G = EW // CG

ET = E_PAD // NS
CS = 1280
RS = CS // 128
OC_S = ET // CS

NPHASE = 2
RSZ = 25088
SPM_ROWS = RSZ + 128
ZT = SPM_ROWS // NS
WT = RSZ // NS

EB = 2048
NB = 4000


@functools.lru_cache(maxsize=None)
def _sc_kernels():
    mesh = plsc.VectorSubcoreMesh(
        core_axis_name="c", subcore_axis_name="s",
        num_cores=NC, num_subcores=NS)
    sc_params = pltpu.CompilerParams(use_tc_tiling_on_sc=False)

    @functools.partial(
        pl.kernel,
        out_type=jax.ShapeDtypeStruct((E_PAD, D), jnp.float32),
        mesh=mesh,
        compiler_params=sc_params,
        scratch_types=[
            pltpu.VMEM((CG,), jnp.int32),
            pltpu.VMEM((CG, D), jnp.float32),
            pltpu.SemaphoreType.DMA,
        ],
    )
    def sc_gather(h_hbm, dst_hbm, out_hbm, idx_v, rows_v, sem):
        wid = lax.axis_index("s") * NC + lax.axis_index("c")

        def round_body(oc, carry):
            base = wid * EW + oc * CG
            pltpu.sync_copy(dst_hbm.at[pl.ds(base, CG)], idx_v)
            cps = [
                pltpu.async_copy(h_hbm.at[idx_v.at[pl.ds(j * 128, 128)]],
                                 rows_v.at[pl.ds(j * 128, 128)], sem)
                for j in range(RG)
            ]
            for cp in cps:
                cp.wait()
            pltpu.sync_copy(rows_v, out_hbm.at[pl.ds(base, CG)])
            return carry

        lax.fori_loop(0, OC_G, round_body, 0)

    @functools.partial(
        pl.kernel,
        out_type=jax.ShapeDtypeStruct((NPHASE * NC, RSZ, D), jnp.float32),
        mesh=mesh,
        compiler_params=sc_params,
        scratch_types=[
            pltpu.VMEM((CS,), jnp.int32),
            pltpu.VMEM((RS, 128), jnp.int32),
            pltpu.VMEM((CS, D), jnp.float32),
            pltpu.VMEM_SHARED((SPM_ROWS, D), jnp.float32),
        ],
    )
    def sc_scatter(t_hbm, src_hbm, zeros_hbm, out_hbm, sidx_v, lidx_v,
                   rows_v, shared):
        core = lax.axis_index("c")
        tid = lax.axis_index("s")

        for phase in range(NPHASE):
            rng = phase * NC + core
            lo = rng * RSZ

            pltpu.sync_copy(zeros_hbm.at[pl.ds(tid * ZT, ZT)],
                            shared.at[pl.ds(tid * ZT, ZT)])
            plsc.subcore_barrier()

            def round_body(oc, carry):
                ebase = tid * ET + oc * CS
                pltpu.sync_copy(src_hbm.at[pl.ds(ebase, CS)], sidx_v)
                pltpu.sync_copy(t_hbm.at[pl.ds(ebase, CS)], rows_v)

                def idx_body(i, carry2):
                    v = sidx_v[pl.ds(i * 16, 16)]
                    li = v - lo
                    ok = (li >= 0) & (li < RSZ)
                    dummy = RSZ + (v & 127)
                    lidx_v[i // 8, pl.ds((i % 8) * 16, 16)] = jnp.where(
                        ok, li, dummy)
                    return carry2

                lax.fori_loop(0, RS * 8, idx_body, 0)
                for j in range(RS):
                    pltpu.sync_copy(rows_v.at[pl.ds(j * 128, 128)],
                                    shared.at[lidx_v.at[j]], add=True)
                return carry

            lax.fori_loop(0, OC_S, round_body, 0)
            plsc.subcore_barrier()
            pltpu.sync_copy(shared.at[pl.ds(tid * WT, WT)],
                            out_hbm.at[rng, pl.ds(tid * WT, WT)])
            plsc.subcore_barrier()

    return sc_gather, sc_scatter


def _bf(x):
    return x.astype(jnp.bfloat16).astype(jnp.float32)


def _transform_body(nbr_ref, ba_ref, w_ref, bl_ref, s_ref, out_ref):
    dn = (((1,), (0,)), ((), ()))
    bm = lax.dot_general(ba_ref[...], w_ref[...], dn,
                         preferred_element_type=jnp.float32) + bl_ref[...]
    z = _bf(bm) * jnp.tile(_bf(nbr_ref[...]), (1, D))
    zh = z.astype(jnp.bfloat16)
    zl = (z - zh.astype(jnp.float32)).astype(jnp.bfloat16)
    s = s_ref[...]
    out_ref[...] = (
        lax.dot_general(zh, s, dn, preferred_element_type=jnp.float32)
        + lax.dot_general(zl, s, dn, preferred_element_type=jnp.float32))


def _tc_transform(nbr, bond16, w2, blin_row, sel, interpret=False):
    return pl.pallas_call(
        _transform_body,
        grid=(E_PAD // EB,),
        in_specs=[
            pl.BlockSpec((EB, D), lambda i: (i, 0)),
            pl.BlockSpec((EB, BD), lambda i: (i, 0)),
            pl.BlockSpec((BD, D * D), lambda i: (0, 0)),
            pl.BlockSpec((1, D * D), lambda i: (0, 0)),
            pl.BlockSpec((D * D, D), lambda i: (0, 0)),
        ],
        out_specs=pl.BlockSpec((EB, D), lambda i: (i, 0)),
        out_shape=jax.ShapeDtypeStruct((E_PAD, D), jnp.float32),
        interpret=interpret,
    )(nbr, bond16, w2, blin_row, sel)


def _exp_precise(x):
    x = jnp.clip(x, -87.0, 87.0)
    k = jnp.round(x * 1.4426950408889634)
    t = x - k * 0.6931471805599453
    p = 1.0 + t * (1.0 + t * (0.5 + t * (
        0.16666666666666666 + t * (0.041666666666666664 + t * (
            0.008333333333333333 + t * (
                0.001388888888888889 + t * 0.0001984126984126984))))))
    scale = lax.bitcast_convert_type(
        (k.astype(jnp.int32) + 127) << 23, jnp.float32)
    return p * scale


def _sigmoid(x):
    return 1.0 / (1.0 + _exp_precise(-x))


def _tanh(x):
    t = _exp_precise(-2.0 * jnp.abs(x))
    return jnp.sign(x) * (1.0 - t) / (1.0 + t)


def _gru_body(a_ref, h_ref, wi_ref, wh_ref, bi_ref, bh_ref, out_ref):
    a = a_ref[...]
    h = h_ref[...]
    gi = lax.dot_general(a.astype(jnp.bfloat16), wi_ref[...],
                         (((1,), (0,)), ((), ())),
                         preferred_element_type=jnp.float32) + bi_ref[...]
    gh = lax.dot_general(h.astype(jnp.bfloat16), wh_ref[...],
                         (((1,), (0,)), ((), ())),
                         preferred_element_type=jnp.float32) + bh_ref[...]
    i_r, i_z, i_n = gi[:, 0:D], gi[:, D:2 * D], gi[:, 2 * D:3 * D]
    h_r, h_z, h_n = gh[:, 0:D], gh[:, D:2 * D], gh[:, 2 * D:3 * D]
    r = _sigmoid(i_r + h_r)
    z = _sigmoid(i_z + h_z)
    n = _tanh(i_n + r * h_n)
    out_ref[...] = (1.0 - z) * n + z * h


def _tc_gru(agg, h, wi, wh, bi, bh, interpret=False):
    return pl.pallas_call(
        _gru_body,
        grid=(N_ATOMS // NB,),
        in_specs=[
            pl.BlockSpec((NB, D), lambda i: (i, 0)),
            pl.BlockSpec((NB, D), lambda i: (i, 0)),
            pl.BlockSpec((D, 3 * D), lambda i: (0, 0)),
            pl.BlockSpec((D, 3 * D), lambda i: (0, 0)),
            pl.BlockSpec((1, 3 * D), lambda i: (0, 0)),
            pl.BlockSpec((1, 3 * D), lambda i: (0, 0)),
        ],
        out_specs=pl.BlockSpec((NB, D), lambda i: (i, 0)),
        out_shape=jax.ShapeDtypeStruct((N_ATOMS, D), jnp.float32),
        interpret=interpret,
    )(agg, h, wi, wh, bi, bh)


def kernel(atom_features, bond_features, pair_indices, W_lin, b_lin,
           W_ih, W_hh, b_ih, b_hh):
    sc_gather, sc_scatter = _sc_kernels()
    src = pair_indices[:, 0]
    dst = pair_indices[:, 1]
    npad = E_PAD - N_EDGES
    dst_pad = jnp.concatenate([dst, jnp.zeros((npad,), jnp.int32)])
    src_pad = jnp.concatenate([src, jnp.full((npad,), 1 << 28, jnp.int32)])
    bond16 = jnp.concatenate([
        bond_features.astype(jnp.bfloat16),
        jnp.zeros((npad, BD), jnp.bfloat16)], axis=0)
    w2 = W_lin.astype(jnp.bfloat16).T
    blin_row = b_lin.reshape(1, D * D)
    sel = jnp.repeat(jnp.eye(D, dtype=jnp.bfloat16), D, axis=0)
    wi = W_ih.T.astype(jnp.bfloat16)
    wh = W_hh.T.astype(jnp.bfloat16)
    bi = b_ih.reshape(1, 3 * D)
    bh = b_hh.reshape(1, 3 * D)
    zeros_buf = jnp.zeros((SPM_ROWS, D), jnp.float32)

    h = atom_features
    for _ in range(N_STEPS):
        nbr = sc_gather(h, dst_pad)
        transformed = _tc_transform(nbr, bond16, w2, blin_row, sel)
        agg4 = sc_scatter(transformed, src_pad, zeros_buf)
        agg = agg4.reshape(NPHASE * NC * RSZ, D)[:N_ATOMS]
        h = _tc_gru(agg, h, wi, wh, bi, bh)
    return h


# speedup vs baseline: 2.0485x; 1.0162x over previous
import functools

import jax
import jax.numpy as jnp
from jax import lax
from jax.experimental import pallas as pl
from jax.experimental.pallas import tpu as pltpu
from jax.experimental.pallas import tpu_sc as plsc

D = 32
BD = 16
KA = BD + 1
N_ATOMS = 100000
N_EDGES = 200000
N_STEPS = 4

NC, NS = 2, 16
NW = NC * NS

E_PAD = 204800
EW = E_PAD // NW
CG = 1280
RG = CG // 128
OC_G = EW // CG

ET = E_PAD // NS
CS = 1280
RS = CS // 128
OC_S = ET // CS

NPHASE = 2
RSZ = 25088
SPM_ROWS = RSZ + 128
ZT = SPM_ROWS // NS
WT = RSZ // NS

EB = 2048
NB = 4000


@functools.lru_cache(maxsize=None)
def _sc_kernels():
    mesh = plsc.VectorSubcoreMesh(
        core_axis_name="c", subcore_axis_name="s",
        num_cores=NC, num_subcores=NS)
    sc_params = pltpu.CompilerParams(use_tc_tiling_on_sc=False)

    @functools.partial(
        pl.kernel,
        out_type=jax.ShapeDtypeStruct((E_PAD, D), jnp.float32),
        mesh=mesh,
        compiler_params=sc_params,
        scratch_types=[
            pltpu.VMEM((CG,), jnp.int32),
            pltpu.VMEM((CG, D), jnp.float32),
            pltpu.SemaphoreType.DMA,
        ],
    )
    def sc_gather(h_hbm, dst_hbm, out_hbm, idx_v, rows_v, sem):
        wid = lax.axis_index("s") * NC + lax.axis_index("c")

        def round_body(oc, carry):
            base = wid * EW + oc * CG
            pltpu.sync_copy(dst_hbm.at[pl.ds(base, CG)], idx_v)
            cps = [
                pltpu.async_copy(h_hbm.at[idx_v.at[pl.ds(j * 128, 128)]],
                                 rows_v.at[pl.ds(j * 128, 128)], sem)
                for j in range(RG)
            ]
            for cp in cps:
                cp.wait()
            pltpu.sync_copy(rows_v, out_hbm.at[pl.ds(base, CG)])
            return carry

        lax.fori_loop(0, OC_G, round_body, 0)

    @functools.partial(
        pl.kernel,
        out_type=jax.ShapeDtypeStruct((NPHASE * NC, RSZ, D), jnp.float32),
        mesh=mesh,
        compiler_params=sc_params,
        scratch_types=[
            pltpu.VMEM((CS,), jnp.int32),
            pltpu.VMEM((RS, 128), jnp.int32),
            pltpu.VMEM((CS, D), jnp.float32),
            pltpu.VMEM_SHARED((SPM_ROWS, D), jnp.float32),
            pltpu.SemaphoreType.DMA,
        ],
    )
    def sc_scatter(t_hbm, src_hbm, zeros_hbm, out_hbm, sidx_v, lidx_v,
                   rows_v, shared, sem):
        core = lax.axis_index("c")
        tid = lax.axis_index("s")

        for phase in range(NPHASE):
            rng = phase * NC + core
            lo = rng * RSZ

            pltpu.sync_copy(zeros_hbm.at[pl.ds(tid * ZT, ZT)],
                            shared.at[pl.ds(tid * ZT, ZT)])
            plsc.subcore_barrier()

            def round_body(oc, carry):
                ebase = tid * ET + oc * CS
                cp_rows = pltpu.async_copy(
                    t_hbm.at[pl.ds(ebase, CS)], rows_v, sem)
                pltpu.sync_copy(src_hbm.at[pl.ds(ebase, CS)], sidx_v)

                def idx_body(i, carry2):
                    v = sidx_v[pl.ds(i * 16, 16)]
                    li = v - lo
                    ok = (li >= 0) & (li < RSZ)
                    dummy = RSZ + (v & 127)
                    lidx_v[i // 8, pl.ds((i % 8) * 16, 16)] = jnp.where(
                        ok, li, dummy)
                    return carry2

                lax.fori_loop(0, RS * 8, idx_body, 0)
                cp_rows.wait()
                for j in range(RS):
                    pltpu.sync_copy(rows_v.at[pl.ds(j * 128, 128)],
                                    shared.at[lidx_v.at[j]], add=True)
                return carry

            lax.fori_loop(0, OC_S, round_body, 0)
            plsc.subcore_barrier()
            pltpu.sync_copy(shared.at[pl.ds(tid * WT, WT)],
                            out_hbm.at[rng, pl.ds(tid * WT, WT)])
            plsc.subcore_barrier()

    return sc_gather, sc_scatter


def _bf(x):
    return x.astype(jnp.bfloat16).astype(jnp.float32)


def _transform_body(nbr_ref, ba_ref, w_ref, bl_ref, s_ref, out_ref):
    dn = (((1,), (0,)), ((), ()))
    bm = lax.dot_general(ba_ref[...], w_ref[...], dn,
                         preferred_element_type=jnp.float32) + bl_ref[...]
    z = _bf(bm) * jnp.tile(_bf(nbr_ref[...]), (1, D))
    zh = z.astype(jnp.bfloat16)
    zl = (z - zh.astype(jnp.float32)).astype(jnp.bfloat16)
    s = s_ref[...]
    out_ref[...] = (
        lax.dot_general(zh, s, dn, preferred_element_type=jnp.float32)
        + lax.dot_general(zl, s, dn, preferred_element_type=jnp.float32))


def _tc_transform(nbr, bond16, w2, blin_row, sel, interpret=False):
    return pl.pallas_call(
        _transform_body,
        grid=(E_PAD // EB,),
        in_specs=[
            pl.BlockSpec((EB, D), lambda i: (i, 0)),
            pl.BlockSpec((EB, BD), lambda i: (i, 0)),
            pl.BlockSpec((BD, D * D), lambda i: (0, 0)),
            pl.BlockSpec((1, D * D), lambda i: (0, 0)),
            pl.BlockSpec((D * D, D), lambda i: (0, 0)),
        ],
        out_specs=pl.BlockSpec((EB, D), lambda i: (i, 0)),
        out_shape=jax.ShapeDtypeStruct((E_PAD, D), jnp.float32),
        interpret=interpret,
    )(nbr, bond16, w2, blin_row, sel)


def _exp_precise(x):
    x = jnp.clip(x, -87.0, 87.0)
    k = jnp.round(x * 1.4426950408889634)
    t = x - k * 0.6931471805599453
    p = 1.0 + t * (1.0 + t * (0.5 + t * (
        0.16666666666666666 + t * (0.041666666666666664 + t * (
            0.008333333333333333 + t * (
                0.001388888888888889 + t * 0.0001984126984126984))))))
    scale = lax.bitcast_convert_type(
        (k.astype(jnp.int32) + 127) << 23, jnp.float32)
    return p * scale


def _sigmoid(x):
    return 1.0 / (1.0 + _exp_precise(-x))


def _tanh(x):
    t = _exp_precise(-2.0 * jnp.abs(x))
    return jnp.sign(x) * (1.0 - t) / (1.0 + t)


def _gru_body(a_ref, h_ref, wi_ref, wh_ref, bi_ref, bh_ref, out_ref):
    a = a_ref[...]
    h = h_ref[...]
    gi = lax.dot_general(a.astype(jnp.bfloat16), wi_ref[...],
                         (((1,), (0,)), ((), ())),
                         preferred_element_type=jnp.float32) + bi_ref[...]
    gh = lax.dot_general(h.astype(jnp.bfloat16), wh_ref[...],
                         (((1,), (0,)), ((), ())),
                         preferred_element_type=jnp.float32) + bh_ref[...]
    i_r, i_z, i_n = gi[:, 0:D], gi[:, D:2 * D], gi[:, 2 * D:3 * D]
    h_r, h_z, h_n = gh[:, 0:D], gh[:, D:2 * D], gh[:, 2 * D:3 * D]
    r = _sigmoid(i_r + h_r)
    z = _sigmoid(i_z + h_z)
    n = _tanh(i_n + r * h_n)
    out_ref[...] = (1.0 - z) * n + z * h


def _tc_gru(agg, h, wi, wh, bi, bh, interpret=False):
    return pl.pallas_call(
        _gru_body,
        grid=(N_ATOMS // NB,),
        in_specs=[
            pl.BlockSpec((NB, D), lambda i: (i, 0)),
            pl.BlockSpec((NB, D), lambda i: (i, 0)),
            pl.BlockSpec((D, 3 * D), lambda i: (0, 0)),
            pl.BlockSpec((D, 3 * D), lambda i: (0, 0)),
            pl.BlockSpec((1, 3 * D), lambda i: (0, 0)),
            pl.BlockSpec((1, 3 * D), lambda i: (0, 0)),
        ],
        out_specs=pl.BlockSpec((NB, D), lambda i: (i, 0)),
        out_shape=jax.ShapeDtypeStruct((N_ATOMS, D), jnp.float32),
        interpret=interpret,
    )(agg, h, wi, wh, bi, bh)


def kernel(atom_features, bond_features, pair_indices, W_lin, b_lin,
           W_ih, W_hh, b_ih, b_hh):
    sc_gather, sc_scatter = _sc_kernels()
    src = pair_indices[:, 0]
    dst = pair_indices[:, 1]
    npad = E_PAD - N_EDGES
    dst_pad = jnp.concatenate([dst, jnp.zeros((npad,), jnp.int32)])
    src_pad = jnp.concatenate([src, jnp.full((npad,), 1 << 28, jnp.int32)])
    bond16 = jnp.concatenate([
        bond_features.astype(jnp.bfloat16),
        jnp.zeros((npad, BD), jnp.bfloat16)], axis=0)
    w2 = W_lin.astype(jnp.bfloat16).T
    blin_row = b_lin.reshape(1, D * D)
    sel = jnp.repeat(jnp.eye(D, dtype=jnp.bfloat16), D, axis=0)
    wi = W_ih.T.astype(jnp.bfloat16)
    wh = W_hh.T.astype(jnp.bfloat16)
    bi = b_ih.reshape(1, 3 * D)
    bh = b_hh.reshape(1, 3 * D)
    zeros_buf = jnp.zeros((SPM_ROWS, D), jnp.float32)

    h = atom_features
    for _ in range(N_STEPS):
        nbr = sc_gather(h, dst_pad)
        transformed = _tc_transform(nbr, bond16, w2, blin_row, sel)
        agg4 = sc_scatter(transformed, src_pad, zeros_buf)
        agg = agg4.reshape(NPHASE * NC * RSZ, D)[:N_ATOMS]
        h = _tc_gru(agg, h, wi, wh, bi, bh)
    return h
